# baseline (speedup 1.0000x reference)
import functools

import jax
import jax.numpy as jnp
from jax import lax
from jax.experimental import pallas as pl
from jax.experimental.pallas import tpu as pltpu
from jax.experimental.pallas import tpu_sc as plsc

_BATCH = 4096
_SEQ = 200
_EMBED = 64
_B = _BATCH * _SEQ

_info = plsc.get_sparse_core_info()
_NC, _NS = _info.num_cores, _info.num_subcores
_NW = _NC * _NS
_PER_W = _B // _NW
_C = 800
_CHUNKS = _PER_W // _C

_mesh = plsc.VectorSubcoreMesh(core_axis_name="c", subcore_axis_name="s")


@functools.partial(
    pl.kernel,
    mesh=_mesh,
    out_type=jax.ShapeDtypeStruct((_B, _EMBED), jnp.float32),
    scratch_types=[
        pltpu.VMEM((_C,), jnp.int32),
        pltpu.VMEM((_C, _EMBED), jnp.float32),
        pltpu.SemaphoreType.DMA,
    ],
    compiler_params=pltpu.CompilerParams(use_tc_tiling_on_sc=False),
)
def _gather_kernel(ids_hbm, table_hbm, out_hbm, idx_v, rows_v, sem):
    wid = lax.axis_index("s") * _NC + lax.axis_index("c")
    base = wid * _PER_W

    def body(g, carry):
        off = base + g * _C
        pltpu.sync_copy(ids_hbm.at[pl.ds(off, _C)], idx_v)
        pltpu.async_copy(table_hbm.at[idx_v], rows_v, sem).wait()
        pltpu.sync_copy(rows_v, out_hbm.at[pl.ds(off, _C)])
        return carry

    lax.fori_loop(0, _CHUNKS, body, 0)


def kernel(input_ids, table):
    ids = input_ids.reshape(-1).astype(jnp.int32)
    out = _gather_kernel(ids, table)
    return out.reshape(_BATCH, _SEQ, _EMBED)


# speedup vs baseline: 1.0194x; 1.0194x over previous
import functools

import jax
import jax.numpy as jnp
from jax import lax
from jax.experimental import pallas as pl
from jax.experimental.pallas import tpu as pltpu
from jax.experimental.pallas import tpu_sc as plsc

_BATCH = 4096
_SEQ = 200
_EMBED = 64
_B = _BATCH * _SEQ

_info = plsc.get_sparse_core_info()
_NC, _NS = _info.num_cores, _info.num_subcores
_NW = _NC * _NS
_PER_W = _B // _NW
_C = 800
_CHUNKS = _PER_W // _C
_PAIRS = _CHUNKS // 2

_mesh = plsc.VectorSubcoreMesh(core_axis_name="c", subcore_axis_name="s")


@functools.partial(
    pl.kernel,
    mesh=_mesh,
    out_type=jax.ShapeDtypeStruct((_B, _EMBED), jnp.float32),
    scratch_types=[
        pltpu.VMEM((_C,), jnp.int32),
        pltpu.VMEM((_C,), jnp.int32),
        pltpu.VMEM((_C, _EMBED), jnp.float32),
        pltpu.VMEM((_C, _EMBED), jnp.float32),
        pltpu.SemaphoreType.DMA,
        pltpu.SemaphoreType.DMA,
        pltpu.SemaphoreType.DMA,
        pltpu.SemaphoreType.DMA,
    ],
    compiler_params=pltpu.CompilerParams(use_tc_tiling_on_sc=False),
)
def _gather_kernel(
    ids_hbm, table_hbm, out_hbm, idx0, idx1, rows0, rows1, g0s, g1s, o0s, o1s
):
    wid = lax.axis_index("s") * _NC + lax.axis_index("c")
    base = wid * _PER_W

    def body(t, carry):
        off0 = base + (2 * t) * _C
        off1 = off0 + _C
        prev0 = off0 - 2 * _C
        prev1 = off1 - 2 * _C

        pltpu.sync_copy(ids_hbm.at[pl.ds(off0, _C)], idx0)

        @pl.when(t > 0)
        def _():
            pltpu.make_async_copy(rows0, out_hbm.at[pl.ds(prev0, _C)], o0s).wait()

        g0 = pltpu.async_copy(table_hbm.at[idx0], rows0, g0s)
        pltpu.sync_copy(ids_hbm.at[pl.ds(off1, _C)], idx1)
        g0.wait()

        @pl.when(t > 0)
        def _():
            pltpu.make_async_copy(rows1, out_hbm.at[pl.ds(prev1, _C)], o1s).wait()

        g1 = pltpu.async_copy(table_hbm.at[idx1], rows1, g1s)
        pltpu.async_copy(rows0, out_hbm.at[pl.ds(off0, _C)], o0s)
        g1.wait()
        pltpu.async_copy(rows1, out_hbm.at[pl.ds(off1, _C)], o1s)
        return carry

    lax.fori_loop(0, _PAIRS, body, 0)

    last0 = base + (_CHUNKS - 2) * _C
    pltpu.make_async_copy(rows0, out_hbm.at[pl.ds(last0, _C)], o0s).wait()
    pltpu.make_async_copy(rows1, out_hbm.at[pl.ds(last0 + _C, _C)], o1s).wait()


def kernel(input_ids, table):
    ids = input_ids.reshape(-1).astype(jnp.int32)
    out = _gather_kernel(ids, table)
    return out.reshape(_BATCH, _SEQ, _EMBED)


# speedup vs baseline: 1.0201x; 1.0006x over previous
import functools

import jax
import jax.numpy as jnp
from jax import lax
from jax.experimental import pallas as pl
from jax.experimental.pallas import tpu as pltpu
from jax.experimental.pallas import tpu_sc as plsc

_BATCH = 4096
_SEQ = 200
_EMBED = 64
_B = _BATCH * _SEQ

_info = plsc.get_sparse_core_info()
_NC, _NS = _info.num_cores, _info.num_subcores
_NW = _NC * _NS
_PER_W = _B // _NW
_NSLOT = 4
_C = 400
_CHUNKS = _PER_W // _C
_ITERS = _CHUNKS // _NSLOT

_mesh = plsc.VectorSubcoreMesh(core_axis_name="c", subcore_axis_name="s")


@functools.partial(
    pl.kernel,
    mesh=_mesh,
    out_type=jax.ShapeDtypeStruct((_B, _EMBED), jnp.float32),
    scratch_types=(
        [pltpu.VMEM((_C,), jnp.int32) for _ in range(_NSLOT)]
        + [pltpu.VMEM((_C, _EMBED), jnp.float32) for _ in range(_NSLOT)]
        + [pltpu.SemaphoreType.DMA for _ in range(3 * _NSLOT)]
    ),
    compiler_params=pltpu.CompilerParams(use_tc_tiling_on_sc=False),
)
def _gather_kernel(ids_hbm, table_hbm, out_hbm, *scr):
    idx = scr[0:_NSLOT]
    rows = scr[_NSLOT : 2 * _NSLOT]
    isem = scr[2 * _NSLOT : 3 * _NSLOT]
    gsem = scr[3 * _NSLOT : 4 * _NSLOT]
    osem = scr[4 * _NSLOT : 5 * _NSLOT]

    wid = lax.axis_index("s") * _NC + lax.axis_index("c")
    base = wid * _PER_W

    for s in range(_NSLOT):
        pltpu.async_copy(ids_hbm.at[pl.ds(base + s * _C, _C)], idx[s], isem[s])

    def body(t, carry):
        off = base + t * (_NSLOT * _C)
        gathers = []
        for s in range(_NSLOT):
            o = off + s * _C
            pltpu.make_async_copy(ids_hbm.at[pl.ds(o, _C)], idx[s], isem[s]).wait()

            @pl.when(t > 0)
            def _(s=s, o=o):
                pltpu.make_async_copy(
                    rows[s], out_hbm.at[pl.ds(o - _NSLOT * _C, _C)], osem[s]
                ).wait()

            gathers.append(pltpu.async_copy(table_hbm.at[idx[s]], rows[s], gsem[s]))

        for s in range(_NSLOT):
            o = off + s * _C
            gathers[s].wait()
            pltpu.async_copy(rows[s], out_hbm.at[pl.ds(o, _C)], osem[s])

            @pl.when(t < _ITERS - 1)
            def _(s=s, o=o):
                pltpu.async_copy(
                    ids_hbm.at[pl.ds(o + _NSLOT * _C, _C)], idx[s], isem[s]
                )

        return carry

    lax.fori_loop(0, _ITERS, body, 0)

    for s in range(_NSLOT):
        o = base + (_CHUNKS - _NSLOT + s) * _C
        pltpu.make_async_copy(rows[s], out_hbm.at[pl.ds(o, _C)], osem[s]).wait()


def kernel(input_ids, table):
    ids = input_ids.reshape(-1).astype(jnp.int32)
    out = _gather_kernel(ids, table)
    return out.reshape(_BATCH, _SEQ, _EMBED)


# speedup vs baseline: 1.0489x; 1.0283x over previous
import functools

import jax
import jax.numpy as jnp
from jax import lax
from jax.experimental import pallas as pl
from jax.experimental.pallas import tpu as pltpu
from jax.experimental.pallas import tpu_sc as plsc

_BATCH = 4096
_SEQ = 200
_EMBED = 64
_B = _BATCH * _SEQ

_info = plsc.get_sparse_core_info()
_NC, _NS = _info.num_cores, _info.num_subcores
_NW = _NC * _NS
_PER_W = _B // _NW
_NSLOT = 4
_C = 400
_CHUNKS = _PER_W // _C
_ITERS = _CHUNKS // _NSLOT

_mesh = plsc.VectorSubcoreMesh(core_axis_name="c", subcore_axis_name="s")


@functools.partial(
    pl.kernel,
    mesh=_mesh,
    out_type=jax.ShapeDtypeStruct((_B, _EMBED), jnp.float32),
    scratch_types=(
        [pltpu.VMEM((_C,), jnp.int32) for _ in range(_NSLOT)]
        + [pltpu.VMEM((_C, _EMBED), jnp.float32) for _ in range(_NSLOT)]
        + [pltpu.SemaphoreType.DMA for _ in range(3 * _NSLOT)]
    ),
    compiler_params=pltpu.CompilerParams(use_tc_tiling_on_sc=False),
)
def _gather_kernel(ids_hbm, table_hbm, out_hbm, *scr):
    idx = scr[0:_NSLOT]
    rows = scr[_NSLOT : 2 * _NSLOT]
    isem = scr[2 * _NSLOT : 3 * _NSLOT]
    gsem = scr[3 * _NSLOT : 4 * _NSLOT]
    osem = scr[4 * _NSLOT : 5 * _NSLOT]

    wid = lax.axis_index("s") * _NC + lax.axis_index("c")
    base = wid * _PER_W

    for s in range(_NSLOT):
        pltpu.async_copy(ids_hbm.at[pl.ds(base + s * _C, _C)], idx[s], isem[s])

    def body(t, carry):
        off = base + t * (_NSLOT * _C)
        gathers = []
        for s in range(_NSLOT):
            o = off + s * _C
            pltpu.make_async_copy(ids_hbm.at[pl.ds(o, _C)], idx[s], isem[s]).wait()

            @pl.when(t > 0)
            def _(s=s, o=o):
                pltpu.make_async_copy(
                    rows[s], out_hbm.at[pl.ds(o - _NSLOT * _C, _C)], osem[s]
                ).wait()

            gathers.append(pltpu.async_copy(table_hbm.at[idx[s]], rows[s], gsem[s]))

        for s in range(_NSLOT):
            o = off + s * _C
            gathers[s].wait()
            pltpu.async_copy(rows[s], out_hbm.at[pl.ds(o, _C)], osem[s])

            @pl.when(t < _ITERS - 1)
            def _(s=s, o=o):
                pltpu.async_copy(
                    ids_hbm.at[pl.ds(o + _NSLOT * _C, _C)], idx[s], isem[s]
                )

        return carry

    lax.fori_loop(0, _ITERS, body, 0)

    for s in range(_NSLOT):
        o = base + (_CHUNKS - _NSLOT + s) * _C
        pltpu.make_async_copy(rows[s], out_hbm.at[pl.ds(o, _C)], osem[s]).wait()


def kernel(input_ids, table):
    ids = input_ids.T.reshape(-1).astype(jnp.int32)
    out = _gather_kernel(ids, table)
    return out.reshape(_SEQ, _BATCH, _EMBED).transpose(1, 0, 2)
